# baseline (speedup 1.0000x reference)
import functools

import jax
import jax.numpy as jnp
from jax import lax
from jax.experimental import pallas as pl
from jax.experimental.pallas import tpu as pltpu
from jax.experimental.pallas import tpu_sc as plsc

B = 2
C = 4
N = 128 * 128 * 128
ROWS_TOT = N // 128
CH_ROWS = 512
NCHUNK = ROWS_TOT // CH_ROWS
K = max(1, int(N * 0.2))
NB = 4096
LANES = 16
NBS = NB * LANES
NBS_R = NBS // 128
NTILES = 16
N_PER_TILE = N // NTILES
SC_CH = 8192
EPS = 1e-6


def _stage1_body(l_ref, t_ref, ce_ref, part_ref):
    l = l_ref[0]
    t = t_ref[0]
    m = jnp.max(l, axis=0)
    e = jnp.exp(l - m[None])
    se = jnp.sum(e, axis=0)
    lse = jnp.log(se) + m
    tmax = jnp.max(t, axis=0)
    sel = jnp.where(t[0] >= tmax, l[0],
          jnp.where(t[1] >= tmax, l[1],
          jnp.where(t[2] >= tmax, l[2], l[3])))
    ce = lse - sel
    ce_ref[0] = ce
    p = e / se[None]
    vals = [jnp.sum(p[c] * t[c]) for c in range(C)]
    vals += [jnp.sum(p[c]) + jnp.sum(t[c]) for c in range(C)]
    vals += [jnp.max(ce), jnp.min(ce)]
    acc = jnp.zeros((16, 128), jnp.float32)
    rid = lax.broadcasted_iota(jnp.int32, (16, 128), 0)
    for i, v in enumerate(vals):
        acc = jnp.where(rid == i, v, acc)
    part_ref[0, 0] = acc


def _stage1(logits4, target4):
    return pl.pallas_call(
        _stage1_body,
        grid=(B, NCHUNK),
        in_specs=[
            pl.BlockSpec((1, C, CH_ROWS, 128), lambda b, j: (b, 0, j, 0)),
            pl.BlockSpec((1, C, CH_ROWS, 128), lambda b, j: (b, 0, j, 0)),
        ],
        out_specs=[
            pl.BlockSpec((1, CH_ROWS, 128), lambda b, j: (b, j, 0)),
            pl.BlockSpec((1, 1, 16, 128), lambda b, j: (b, j, 0, 0)),
        ],
        out_shape=[
            jax.ShapeDtypeStruct((B, ROWS_TOT, 128), jnp.float32),
            jax.ShapeDtypeStruct((B, NCHUNK, 16, 128), jnp.float32),
        ],
    )(logits4, target4)


def _sc_hist(ce2, partials):
    mesh = plsc.VectorSubcoreMesh(core_axis_name="c", subcore_axis_name="s")

    @functools.partial(
        pl.kernel,
        mesh=mesh,
        out_type=[
            jax.ShapeDtypeStruct((B, NTILES, NB * LANES), jnp.float32),
        ],
        scratch_types=[
            pltpu.VMEM((NCHUNK, 2, 128), jnp.float32),
            pltpu.VMEM((SC_CH,), jnp.float32),
            pltpu.VMEM((SC_CH,), jnp.float32),
            pltpu.VMEM((NB * LANES,), jnp.float32),
            pltpu.SemaphoreType.DMA,
            pltpu.SemaphoreType.DMA,
        ],
        compiler_params=pltpu.CompilerParams(needs_layout_passes=False),
    )
    def k(ce_hbm, part_hbm, cnt_hbm, slab, buf0, buf1, cnt, sem0, sem1):
        c = lax.axis_index("c")
        s = lax.axis_index("s")
        sample_off = c * N
        pltpu.sync_copy(part_hbm.at[c, :, pl.ds(8, 2), :], slab)
        lo = jnp.full((16,), 3.4e38, jnp.float32)
        hi = -lo
        for j in range(NCHUNK):
            hi = jnp.maximum(hi, slab[j, 0, pl.ds(0, 16)])
            lo = jnp.minimum(lo, slab[j, 1, pl.ds(0, 16)])
        scale = jnp.full((16,), float(NB), jnp.float32) / jnp.maximum(
            hi - lo, jnp.full((16,), 1e-30, jnp.float32))
        zeros = jnp.zeros((16,), jnp.float32)

        def zbody(i, carry):
            for u in range(8):
                cnt[pl.ds((i * 8 + u) * 16, 16)] = zeros
            return carry

        lax.fori_loop(0, NB * LANES // 128, zbody, 0)

        ones = jnp.full((16,), 1.0, jnp.float32)
        nbm1 = jnp.full((16,), NB - 1, jnp.int32)
        lane = lax.broadcasted_iota(jnp.int32, (16,), 0)
        base = s * N_PER_TILE
        nch = N_PER_TILE // SC_CH

        def process(bref):
            def grp(g, carry2):
                for u in range(8):
                    x = bref[pl.ds(g * 128 + u * 16, 16)]
                    idx = jnp.minimum(((x - lo) * scale).astype(jnp.int32),
                                      nbm1)
                    plsc.addupdate_scatter(cnt, [idx * LANES + lane], ones)
                return carry2

            lax.fori_loop(0, SC_CH // 128, grp, 0)

        def outer(i, carry):
            pltpu.sync_copy(
                ce_hbm.at[pl.ds(sample_off + base + i * SC_CH, SC_CH)], buf0)
            process(buf0)
            return carry

        lax.fori_loop(0, nch, outer, 0)
        pltpu.sync_copy(cnt, cnt_hbm.at[c, s])

    return k(ce2, partials)


def _final_body(part_ref, cnt_ref, out_ref):
    pt = part_ref[...]
    num = jnp.sum(pt[:, :, 0:4, 0:1], axis=(1, 3))
    den = jnp.sum(pt[:, :, 4:8, 0:1], axis=(1, 3))
    dice = 1.0 - 2.0 * num / (den + EPS)
    dice_loss = jnp.mean(dice[:, 1:])
    ce_hi = jnp.max(pt[:, :, 8:9, 0:1], axis=(1, 3))
    ce_lo = jnp.min(pt[:, :, 9:10, 0:1], axis=(1, 3))
    ct = jnp.sum(cnt_ref[...], axis=1)
    jrow = lax.broadcasted_iota(jnp.int32, (NBS_R, 128), 0)
    jcol = lax.broadcasted_iota(jnp.int32, (NBS_R, 128), 1)
    jbin = ((jrow * 128 + jcol) // LANES).astype(jnp.float32)
    width = jnp.maximum(ce_hi - ce_lo, 1e-30) / float(NB)
    mid = (ce_lo[:, :, None]
           + (jbin + 0.5)[None] * width[:, :, None])
    st = ct * mid
    rs = jnp.sum(ct, axis=2)
    a0 = lax.broadcasted_iota(jnp.int32, (NBS_R, NBS_R), 0)
    a1 = lax.broadcasted_iota(jnp.int32, (NBS_R, NBS_R), 1)
    u = (a0 > a1).astype(jnp.float32)
    b0 = lax.broadcasted_iota(jnp.int32, (128, 128), 0)
    b1 = lax.broadcasted_iota(jnp.int32, (128, 128), 1)
    v = (b0 >= b1).astype(jnp.float32)
    hi = lax.Precision.HIGHEST
    srow = jnp.dot(rs, u, precision=hi)
    isuf = jnp.dot(ct, v, precision=hi)
    cum = srow[:, :, None] + isuf
    kf = jnp.float32(K)
    full = (cum < kf).astype(jnp.float32)
    bnd = ((cum >= kf) & (cum - ct < kf)).astype(jnp.float32)
    count_full = jnp.sum(full * ct, axis=(1, 2))
    sum_full = jnp.sum(full * st, axis=(1, 2))
    mid_b = jnp.sum(bnd * mid, axis=(1, 2))
    rem = kf - count_full
    topk_sum = sum_full + rem * mid_b
    topk_loss = jnp.mean(topk_sum) / K
    loss = topk_loss + 0.5 * dice_loss
    out_ref[...] = loss[None, None]


def _final(partials, counts):
    return pl.pallas_call(
        _final_body,
        out_shape=jax.ShapeDtypeStruct((1, 1), jnp.float32),
    )(partials, counts)


def kernel(logits, target):
    logits4 = logits.reshape(B, C, ROWS_TOT, 128)
    target4 = target.reshape(B, C, ROWS_TOT, 128)
    ce, partials = _stage1(logits4, target4)
    (counts,) = _sc_hist(ce.reshape(B * N), partials)
    out = _final(partials, counts.reshape(B, NTILES, NBS_R, 128))
    return out[0, 0]


# speedup vs baseline: 1.1056x; 1.1056x over previous
import functools

import jax
import jax.numpy as jnp
from jax import lax
from jax.experimental import pallas as pl
from jax.experimental.pallas import tpu as pltpu
from jax.experimental.pallas import tpu_sc as plsc

B = 2
C = 4
N = 128 * 128 * 128
ROWS_TOT = N // 128
CH_ROWS = 512
NCHUNK = ROWS_TOT // CH_ROWS
K = max(1, int(N * 0.2))
NB = 8192
NB_R = NB // 128
NTILES = 16
N_PER_TILE = N // NTILES
SC_CH = 16384
EPS = 1e-6


def _stage1_body(l_ref, t_ref, ce_ref, part_ref):
    l = l_ref[0]
    t = t_ref[0]
    m = jnp.max(l, axis=0)
    e = jnp.exp(l - m[None])
    se = jnp.sum(e, axis=0)
    lse = jnp.log(se) + m
    tmax = jnp.max(t, axis=0)
    sel = jnp.where(t[0] >= tmax, l[0],
          jnp.where(t[1] >= tmax, l[1],
          jnp.where(t[2] >= tmax, l[2], l[3])))
    ce = lse - sel
    ce_ref[0] = ce
    p = e / se[None]
    vals = [jnp.sum(p[c] * t[c]) for c in range(C)]
    vals += [jnp.sum(p[c]) + jnp.sum(t[c]) for c in range(C)]
    vals += [jnp.max(ce), jnp.min(ce)]
    acc = jnp.zeros((16, 128), jnp.float32)
    rid = lax.broadcasted_iota(jnp.int32, (16, 128), 0)
    for i, v in enumerate(vals):
        acc = jnp.where(rid == i, v, acc)
    part_ref[0, 0] = acc


def _stage1(logits4, target4):
    return pl.pallas_call(
        _stage1_body,
        grid=(B, NCHUNK),
        in_specs=[
            pl.BlockSpec((1, C, CH_ROWS, 128), lambda b, j: (b, 0, j, 0)),
            pl.BlockSpec((1, C, CH_ROWS, 128), lambda b, j: (b, 0, j, 0)),
        ],
        out_specs=[
            pl.BlockSpec((1, CH_ROWS, 128), lambda b, j: (b, j, 0)),
            pl.BlockSpec((1, 1, 16, 128), lambda b, j: (b, j, 0, 0)),
        ],
        out_shape=[
            jax.ShapeDtypeStruct((B, ROWS_TOT, 128), jnp.float32),
            jax.ShapeDtypeStruct((B, NCHUNK, 16, 128), jnp.float32),
        ],
    )(logits4, target4)


def _sc_hist(ce2, partials):
    mesh = plsc.VectorSubcoreMesh(core_axis_name="c", subcore_axis_name="s")

    @functools.partial(
        pl.kernel,
        mesh=mesh,
        out_type=[
            jax.ShapeDtypeStruct((B, NTILES, NB), jnp.float32),
        ],
        scratch_types=[
            pltpu.VMEM((NCHUNK, 2, 128), jnp.float32),
            pltpu.VMEM((SC_CH,), jnp.float32),
            pltpu.VMEM((SC_CH,), jnp.float32),
            pltpu.VMEM((NB,), jnp.float32),
            pltpu.SemaphoreType.DMA,
            pltpu.SemaphoreType.DMA,
        ],
        compiler_params=pltpu.CompilerParams(needs_layout_passes=False),
    )
    def k(ce_hbm, part_hbm, cnt_hbm, slab, buf0, buf1, cnt, sem0, sem1):
        c = lax.axis_index("c")
        s = lax.axis_index("s")
        sample_off = c * N
        pltpu.sync_copy(part_hbm.at[c, :, pl.ds(8, 2), :], slab)
        lo = jnp.full((16,), 3.4e38, jnp.float32)
        hi = -lo
        for j in range(NCHUNK):
            hi = jnp.maximum(hi, slab[j, 0, pl.ds(0, 16)])
            lo = jnp.minimum(lo, slab[j, 1, pl.ds(0, 16)])
        scale = jnp.full((16,), float(NB), jnp.float32) / jnp.maximum(
            hi - lo, jnp.full((16,), 1e-30, jnp.float32))
        zeros = jnp.zeros((16,), jnp.float32)

        def zbody(i, carry):
            cnt[pl.ds(i * 16, 16)] = zeros
            return carry

        lax.fori_loop(0, NB // 16, zbody, 0)

        ones = jnp.full((16,), 1.0, jnp.float32)
        nbm1 = jnp.full((16,), NB - 1, jnp.int32)
        base = s * N_PER_TILE
        nch = N_PER_TILE // SC_CH

        def process(bref):
            def grp(g, carry2):
                for u in range(8):
                    x = bref[pl.ds(g * 128 + u * 16, 16)]
                    idx = jnp.minimum(((x - lo) * scale).astype(jnp.int32),
                                      nbm1)
                    plsc.addupdate_scatter(cnt, [idx], ones)
                return carry2

            lax.fori_loop(0, SC_CH // 128, grp, 0)

        def outer(i, carry):
            pltpu.sync_copy(
                ce_hbm.at[pl.ds(sample_off + base + i * SC_CH, SC_CH)], buf0)
            process(buf0)
            return carry

        lax.fori_loop(0, nch, outer, 0)
        pltpu.sync_copy(cnt, cnt_hbm.at[c, s])

    return k(ce2, partials)


def _final_body(part_ref, cnt_ref, out_ref):
    pt = part_ref[...]
    num = jnp.sum(pt[:, :, 0:4, 0:1], axis=(1, 3))
    den = jnp.sum(pt[:, :, 4:8, 0:1], axis=(1, 3))
    dice = 1.0 - 2.0 * num / (den + EPS)
    dice_loss = jnp.mean(dice[:, 1:])
    ce_hi = jnp.max(pt[:, :, 8:9, 0:1], axis=(1, 3))
    ce_lo = jnp.min(pt[:, :, 9:10, 0:1], axis=(1, 3))
    ct = jnp.sum(cnt_ref[...], axis=1)
    jrow = lax.broadcasted_iota(jnp.int32, (NB_R, 128), 0)
    jcol = lax.broadcasted_iota(jnp.int32, (NB_R, 128), 1)
    jglob = (jrow * 128 + jcol).astype(jnp.float32)
    width = jnp.maximum(ce_hi - ce_lo, 1e-30) / float(NB)
    mid = (ce_lo[:, :, None]
           + (jglob + 0.5)[None] * width[:, :, None])
    st = ct * mid
    rs = jnp.sum(ct, axis=2)
    a0 = lax.broadcasted_iota(jnp.int32, (NB_R, NB_R), 0)
    a1 = lax.broadcasted_iota(jnp.int32, (NB_R, NB_R), 1)
    u = (a0 > a1).astype(jnp.float32)
    b0 = lax.broadcasted_iota(jnp.int32, (128, 128), 0)
    b1 = lax.broadcasted_iota(jnp.int32, (128, 128), 1)
    v = (b0 >= b1).astype(jnp.float32)
    hi = lax.Precision.HIGHEST
    srow = jnp.dot(rs, u, precision=hi)
    isuf = jnp.dot(ct, v, precision=hi)
    cum = srow[:, :, None] + isuf
    kf = jnp.float32(K)
    full = (cum < kf).astype(jnp.float32)
    bnd = ((cum >= kf) & (cum - ct < kf)).astype(jnp.float32)
    count_full = jnp.sum(full * ct, axis=(1, 2))
    sum_full = jnp.sum(full * st, axis=(1, 2))
    mid_b = jnp.sum(bnd * mid, axis=(1, 2))
    rem = kf - count_full
    topk_sum = sum_full + rem * mid_b
    topk_loss = jnp.mean(topk_sum) / K
    loss = topk_loss + 0.5 * dice_loss
    out_ref[...] = loss[None, None]


def _final(partials, counts):
    return pl.pallas_call(
        _final_body,
        out_shape=jax.ShapeDtypeStruct((1, 1), jnp.float32),
    )(partials, counts)


def kernel(logits, target):
    logits4 = logits.reshape(B, C, ROWS_TOT, 128)
    target4 = target.reshape(B, C, ROWS_TOT, 128)
    ce, partials = _stage1(logits4, target4)
    (counts,) = _sc_hist(ce.reshape(B * N), partials)
    out = _final(partials, counts.reshape(B, NTILES, NB_R, 128))
    return out[0, 0]
